# Initial kernel scaffold; baseline (speedup 1.0000x reference)
import jax
import jax.numpy as jnp
from jax.experimental import pallas as pl


def kernel(x, pos_idx, embedding_weight):
    raise NotImplementedError("write your pallas kernel here")


# speedup vs baseline: 1.6188x; 1.6188x over previous
import functools

import jax
import jax.numpy as jnp
from jax import lax
from jax.experimental import pallas as pl
from jax.experimental.pallas import tpu as pltpu
from jax.experimental.pallas import tpu_sc as plsc

BATCH = 1024
MEM_LEN = 200
POS_DIM = 128
MAX_SEQ_LEN = 48

ROWS = BATCH * MEM_LEN
NUM_WORKERS = 32
ROWS_PER_WORKER = ROWS // NUM_WORKERS
CHUNK = 256
NUM_CHUNKS = ROWS_PER_WORKER // CHUNK
LANES = 16


def _sc_body(x_hbm, idx_hbm, tab_hbm, out_hbm, idx_v, x_v, emb_v, sem_x, sem_e):
    wid = lax.axis_index("s") * 2 + lax.axis_index("c")
    w_base = wid * ROWS_PER_WORKER

    def chunk_body(ci, carry):
        base = w_base + ci * CHUNK
        pltpu.sync_copy(idx_hbm.at[pl.ds(base, CHUNK)], idx_v)
        cp_x = pltpu.async_copy(x_hbm.at[pl.ds(base, CHUNK), :], x_v, sem_x)
        cp_e = pltpu.async_copy(tab_hbm.at[idx_v], emb_v, sem_e)
        cp_x.wait()
        cp_e.wait()

        def row_body(r, c2):
            for j in range(POS_DIM // LANES):
                sl = pl.ds(j * LANES, LANES)
                x_v[r, sl] = x_v[r, sl] + emb_v[r, sl]
            return c2

        lax.fori_loop(0, CHUNK, row_body, 0, unroll=2)
        pltpu.sync_copy(x_v, out_hbm.at[pl.ds(base, CHUNK), :])
        return carry

    lax.fori_loop(0, NUM_CHUNKS, chunk_body, 0)


@jax.jit
def _run(x_flat, idx_flat, table):
    mesh = plsc.VectorSubcoreMesh(core_axis_name="c", subcore_axis_name="s")
    return pl.kernel(
        _sc_body,
        out_type=jax.ShapeDtypeStruct((ROWS, POS_DIM), jnp.float32),
        mesh=mesh,
        scratch_types=[
            pltpu.VMEM((CHUNK,), jnp.int32),
            pltpu.VMEM((CHUNK, POS_DIM), jnp.float32),
            pltpu.VMEM((CHUNK, POS_DIM), jnp.float32),
            pltpu.SemaphoreType.DMA,
            pltpu.SemaphoreType.DMA,
        ],
    )(x_flat, idx_flat, table)


def kernel(x, pos_idx, embedding_weight):
    x_flat = x.reshape(ROWS, POS_DIM)
    idx_flat = pos_idx.reshape(ROWS).astype(jnp.int32)
    out = _run(x_flat, idx_flat, embedding_weight)
    return out.reshape(BATCH, MEM_LEN, POS_DIM)


# speedup vs baseline: 5.7994x; 3.5825x over previous
import functools

import jax
import jax.numpy as jnp
from jax import lax
from jax.experimental import pallas as pl
from jax.experimental.pallas import tpu as pltpu
from jax.experimental.pallas import tpu_sc as plsc

BATCH = 1024
MEM_LEN = 200
POS_DIM = 128
MAX_SEQ_LEN = 48

ROWS = BATCH * MEM_LEN
NUM_WORKERS = 32
ROWS_PER_WORKER = ROWS // NUM_WORKERS
CHUNK = 640
NUM_CHUNKS = ROWS_PER_WORKER // CHUNK


def _sc_body(x_hbm, idx_hbm, tab_hbm, out_hbm, tab_sh, idx_v, x_v):
    cid = lax.axis_index("c")
    sid = lax.axis_index("s")
    wid = sid * 2 + cid
    w_base = wid * ROWS_PER_WORKER

    @pl.when(sid == 0)
    def _stage_table():
        pltpu.sync_copy(tab_hbm, tab_sh)

    plsc.subcore_barrier()

    def chunk_body(ci, carry):
        base = w_base + ci * CHUNK
        pltpu.sync_copy(idx_hbm.at[pl.ds(base, CHUNK)], idx_v)
        pltpu.sync_copy(x_hbm.at[pl.ds(base, CHUNK), :], x_v)
        pltpu.sync_copy(tab_sh.at[idx_v], x_v, add=True)
        pltpu.sync_copy(x_v, out_hbm.at[pl.ds(base, CHUNK), :])
        return carry

    lax.fori_loop(0, NUM_CHUNKS, chunk_body, 0)


@jax.jit
def _run(x_flat, idx_flat, table):
    mesh = plsc.VectorSubcoreMesh(core_axis_name="c", subcore_axis_name="s")
    return pl.kernel(
        _sc_body,
        out_type=jax.ShapeDtypeStruct((ROWS, POS_DIM), jnp.float32),
        mesh=mesh,
        scratch_types=[
            pltpu.VMEM_SHARED((MAX_SEQ_LEN, POS_DIM), jnp.float32),
            pltpu.VMEM((CHUNK,), jnp.int32),
            pltpu.VMEM((CHUNK, POS_DIM), jnp.float32),
        ],
    )(x_flat, idx_flat, table)


def kernel(x, pos_idx, embedding_weight):
    x_flat = x.reshape(ROWS, POS_DIM)
    idx_flat = pos_idx.reshape(ROWS).astype(jnp.int32)
    out = _run(x_flat, idx_flat, embedding_weight)
    return out.reshape(BATCH, MEM_LEN, POS_DIM)


# speedup vs baseline: 7.9075x; 1.3635x over previous
import functools

import jax
import jax.numpy as jnp
from jax import lax
from jax.experimental import pallas as pl
from jax.experimental.pallas import tpu as pltpu
from jax.experimental.pallas import tpu_sc as plsc

BATCH = 1024
MEM_LEN = 200
POS_DIM = 128
MAX_SEQ_LEN = 48

ROWS = BATCH * MEM_LEN
NUM_WORKERS = 32
ROWS_PER_WORKER = ROWS // NUM_WORKERS
CHUNK = 400
NUM_CHUNKS = ROWS_PER_WORKER // CHUNK


def _sc_body(x_hbm, idx_hbm, tab_hbm, out_hbm, tab_sh,
             idx_v0, idx_v1, x_v0, x_v1, sem_i0, sem_i1, sem_o0, sem_o1):
    cid = lax.axis_index("c")
    sid = lax.axis_index("s")
    wid = sid * 2 + cid
    w_base = wid * ROWS_PER_WORKER

    idx_bufs = (idx_v0, idx_v1)
    x_bufs = (x_v0, x_v1)
    sem_in = (sem_i0, sem_i1)
    sem_out = (sem_o0, sem_o1)

    def issue_in(ci, b):
        base = w_base + ci * CHUNK
        d1 = pltpu.async_copy(idx_hbm.at[pl.ds(base, CHUNK)], idx_bufs[b],
                              sem_in[b])
        d2 = pltpu.async_copy(x_hbm.at[pl.ds(base, CHUNK), :], x_bufs[b],
                              sem_in[b])
        return (d1, d2)

    pend_in = [None, None]
    pend_out = [None, None]
    pend_in[0] = issue_in(0, 0)

    @pl.when(sid == 0)
    def _stage_table():
        pltpu.sync_copy(tab_hbm, tab_sh)

    plsc.subcore_barrier()

    for ci in range(NUM_CHUNKS):
        b = ci & 1
        nb = 1 - b
        if ci + 1 < NUM_CHUNKS:
            if pend_out[nb] is not None:
                pend_out[nb].wait()
                pend_out[nb] = None
            pend_in[nb] = issue_in(ci + 1, nb)
        for d in pend_in[b]:
            d.wait()
        pend_in[b] = None
        pltpu.sync_copy(tab_sh.at[idx_bufs[b]], x_bufs[b], add=True)
        base = w_base + ci * CHUNK
        pend_out[b] = pltpu.async_copy(
            x_bufs[b], out_hbm.at[pl.ds(base, CHUNK), :], sem_out[b])

    for b in range(2):
        if pend_out[b] is not None:
            pend_out[b].wait()


@jax.jit
def _run(x_flat, idx_flat, table):
    mesh = plsc.VectorSubcoreMesh(core_axis_name="c", subcore_axis_name="s")
    return pl.kernel(
        _sc_body,
        out_type=jax.ShapeDtypeStruct((ROWS, POS_DIM), jnp.float32),
        mesh=mesh,
        scratch_types=[
            pltpu.VMEM_SHARED((MAX_SEQ_LEN, POS_DIM), jnp.float32),
            pltpu.VMEM((CHUNK,), jnp.int32),
            pltpu.VMEM((CHUNK,), jnp.int32),
            pltpu.VMEM((CHUNK, POS_DIM), jnp.float32),
            pltpu.VMEM((CHUNK, POS_DIM), jnp.float32),
            pltpu.SemaphoreType.DMA,
            pltpu.SemaphoreType.DMA,
            pltpu.SemaphoreType.DMA,
            pltpu.SemaphoreType.DMA,
        ],
    )(x_flat, idx_flat, table)


def kernel(x, pos_idx, embedding_weight):
    x_flat = x.reshape(ROWS, POS_DIM)
    idx_flat = pos_idx.reshape(ROWS).astype(jnp.int32)
    out = _run(x_flat, idx_flat, embedding_weight)
    return out.reshape(BATCH, MEM_LEN, POS_DIM)
